# baseline (speedup 1.0000x reference)
import functools

import jax
import jax.numpy as jnp
from jax.experimental import pallas as pl
from jax.experimental.pallas import tpu as pltpu

E = 8
K = 2
BLK = 256
NEG = -1e30


def _router_kernel(x_ref, lnw_ref, lnb_ref, wg_ref,
                   xn_ref, topi_ref, gates_ref, aux_ref, bal_ref):
    x = x_ref[...]
    mu = jnp.mean(x, axis=-1, keepdims=True)
    var = jnp.mean((x - mu) ** 2, axis=-1, keepdims=True)
    xn = (x - mu) / jnp.sqrt(var + 1e-6) * lnw_ref[...] + lnb_ref[...]
    xn_ref[...] = xn
    logits = jnp.dot(xn, wg_ref[...], preferred_element_type=jnp.float32)
    T = logits.shape[0]
    iota = jax.lax.broadcasted_iota(jnp.int32, (T, E), 1)
    m1 = jnp.max(logits, axis=-1, keepdims=True)
    i1 = jnp.min(jnp.where(logits == m1, iota, E), axis=-1)
    masked = jnp.where(iota == i1[:, None], NEG, logits)
    m2 = jnp.max(masked, axis=-1, keepdims=True)
    i2 = jnp.min(jnp.where(masked == m2, iota, E), axis=-1)
    topi_ref[...] = jnp.stack([i1, i2], axis=-1)
    e2 = jnp.exp(m2 - m1)[:, 0]
    g1 = 1.0 / (1.0 + e2)
    gates_ref[...] = jnp.stack([g1, 1.0 - g1], axis=-1)
    ex = jnp.exp(logits - m1)
    sumex = jnp.sum(ex, axis=-1, keepdims=True)
    probs = ex / sumex
    P = jnp.mean(probs, axis=0)
    dens = jnp.mean((i1[:, None] == iota[:1]).astype(jnp.float32)
                    + (i2[:, None] == iota[:1]).astype(jnp.float32), axis=0)
    aux_ref[...] = (E * jnp.sum(dens * P)).reshape(1, 1)
    z = m1[:, 0] + jnp.log(sumex[:, 0])
    bal_ref[...] = jnp.mean(z * z).reshape(1, 1)


def _expert_kernel(te_ref, tv_ref, xg_ref, w1_ref, b1_ref, w2_ref, b2_ref,
                   out_ref):
    i = pl.program_id(0)

    @pl.when(tv_ref[i] > 0)
    def _():
        h = jnp.dot(xg_ref[...], w1_ref[0], preferred_element_type=jnp.float32)
        h = jax.nn.gelu(h + b1_ref[0])
        y = jnp.dot(h, w2_ref[0], preferred_element_type=jnp.float32)
        out_ref[...] = y + b2_ref[0]


@functools.partial(jax.jit, static_argnames=("interpret",))
def kernel(x_img, ln_w, ln_b, Wg, W1, b1, W2, b2, interpret=False):
    Bb, S, C = x_img.shape
    T = Bb * S
    H = W2.shape[-1]
    TK = T * K
    NP = TK + E * BLK
    NT = NP // BLK
    x = x_img.reshape(T, C)

    xn, topi, gates, aux, bal = pl.pallas_call(
        _router_kernel,
        out_shape=[
            jax.ShapeDtypeStruct((T, C), jnp.float32),
            jax.ShapeDtypeStruct((T, K), jnp.int32),
            jax.ShapeDtypeStruct((T, K), jnp.float32),
            jax.ShapeDtypeStruct((1, 1), jnp.float32),
            jax.ShapeDtypeStruct((1, 1), jnp.float32),
        ],
        interpret=interpret,
    )(x, ln_w.reshape(1, C), ln_b.reshape(1, C), Wg)

    e_flat = topi.reshape(-1)
    oh = (e_flat[:, None] == jnp.arange(E)[None, :]).astype(jnp.int32)
    counts = jnp.sum(oh, axis=0)
    rank = jnp.sum(jnp.cumsum(oh, axis=0) * oh, axis=1) - 1
    padded = ((counts + BLK - 1) // BLK) * BLK
    pad_end = jnp.cumsum(padded)
    pad_off = pad_end - padded
    dest = pad_off[e_flat] + rank
    src = jnp.full((NP,), T, jnp.int32).at[dest].set(
        jnp.arange(TK, dtype=jnp.int32) // K)
    tile_start = jnp.arange(NT, dtype=jnp.int32) * BLK
    te = jnp.searchsorted(pad_end, tile_start, side="right").astype(jnp.int32)
    tv = (tile_start < pad_end[-1]).astype(jnp.int32)
    te = jnp.minimum(te, E - 1)

    xn_pad = jnp.concatenate([xn, jnp.zeros((1, C), jnp.float32)], axis=0)
    xg = jnp.take(xn_pad, src, axis=0)

    grid_spec = pltpu.PrefetchScalarGridSpec(
        num_scalar_prefetch=2,
        grid=(NT,),
        in_specs=[
            pl.BlockSpec((BLK, C), lambda i, te, tv: (i, 0)),
            pl.BlockSpec((1, C, H), lambda i, te, tv: (te[i], 0, 0)),
            pl.BlockSpec((1, 1, H), lambda i, te, tv: (te[i], 0, 0)),
            pl.BlockSpec((1, H, H), lambda i, te, tv: (te[i], 0, 0)),
            pl.BlockSpec((1, 1, H), lambda i, te, tv: (te[i], 0, 0)),
        ],
        out_specs=pl.BlockSpec((BLK, H), lambda i, te, tv: (i, 0)),
    )
    y_pad = pl.pallas_call(
        _expert_kernel,
        grid_spec=grid_spec,
        out_shape=jax.ShapeDtypeStruct((NP, H), jnp.float32),
        interpret=interpret,
    )(te, tv, xg, W1, b1.reshape(E, 1, H), W2, b2.reshape(E, 1, H))

    y_tok = jnp.take(y_pad, dest, axis=0)
    out = jnp.sum((y_tok * gates.reshape(TK, 1)).reshape(T, K, H), axis=1)

    results = out.reshape(Bb, S, H)
    id_experts = topi.reshape(Bb, S, K)
    return results, aux[0, 0], id_experts, bal[0, 0]
